# baseline (speedup 1.0000x reference)
import jax
import jax.numpy as jnp
from jax.experimental import pallas as pl
from jax.experimental.pallas import tpu as pltpu

_B = 16384
_D = 32
_NCB = 8
_K = 1024
_BLK = 2048


def _rvq_body(z_ref, cb_ref, q_ref, idx_ref):
    r = z_ref[...]
    q = jnp.zeros_like(r)
    iota_k = jax.lax.broadcasted_iota(jnp.int32, (_BLK, _K), 1)
    idx_cols = []
    for i in range(_NCB):
        cb = cb_ref[i]
        c2 = jnp.sum(cb * cb, axis=1)
        prod = jax.lax.dot_general(
            r.astype(jnp.bfloat16), cb.astype(jnp.bfloat16), (((1,), (1,)), ((), ())),
            preferred_element_type=jnp.float32,
        )
        r2 = jnp.sum(r * r, axis=1, keepdims=True)
        d = r2 + c2[None, :] - 2.0 * prod
        dmin = jnp.min(d, axis=1, keepdims=True)
        idx = jnp.min(jnp.where(d == dmin, iota_k, _K), axis=1, keepdims=True)
        onehot = (iota_k == idx).astype(jnp.float32)
        g = jax.lax.dot_general(
            onehot, cb, (((1,), (0,)), ((), ())),
            preferred_element_type=jnp.float32,
            precision=jax.lax.Precision.HIGHEST,
        )
        r = r - g
        q = q + g
        idx_cols.append(idx)
    q_ref[...] = q
    idx_ref[...] = jnp.concatenate(idx_cols, axis=1)


def kernel(z, codebooks):
    grid = (_B // _BLK,)
    q, idx = pl.pallas_call(
        _rvq_body,
        grid=grid,
        in_specs=[
            pl.BlockSpec((_BLK, _D), lambda i: (i, 0)),
            pl.BlockSpec((_NCB, _K, _D), lambda i: (0, 0, 0)),
        ],
        out_specs=[
            pl.BlockSpec((_BLK, _D), lambda i: (i, 0)),
            pl.BlockSpec((_BLK, _NCB), lambda i: (i, 0)),
        ],
        out_shape=[
            jax.ShapeDtypeStruct((_B, _D), jnp.float32),
            jax.ShapeDtypeStruct((_B, _NCB), jnp.int32),
        ],
    )(z, codebooks)
    loss = jnp.zeros((), dtype=jnp.float32)
    return q, loss, idx.astype(jnp.int64)


# speedup vs baseline: 2.4911x; 2.4911x over previous
import jax
import jax.numpy as jnp
from jax.experimental import pallas as pl
from jax.experimental.pallas import tpu as pltpu

_B = 16384
_D = 32
_NCB = 8
_K = 1024
_BLK = 2048


def _rvq_body(z_ref, cb_ref, q_ref, idx_ref):
    r = z_ref[...]
    q = jnp.zeros_like(r)
    iota_k = jax.lax.broadcasted_iota(jnp.int32, (_BLK, _K), 1)
    idx_cols = []
    for i in range(_NCB):
        cb = cb_ref[i]
        c2 = jnp.sum(cb * cb, axis=1)
        prod = jax.lax.dot_general(
            r.astype(jnp.bfloat16), cb.astype(jnp.bfloat16), (((1,), (1,)), ((), ())),
            preferred_element_type=jnp.float32,
        )
        r2 = jnp.sum(r * r, axis=1, keepdims=True)
        d = r2 + c2[None, :] - 2.0 * prod
        dmin = jnp.min(d, axis=1, keepdims=True)
        idx = jnp.min(jnp.where(d == dmin, iota_k, _K), axis=1, keepdims=True)
        onehot = (iota_k == idx).astype(jnp.bfloat16)
        p1 = cb.astype(jnp.bfloat16)
        r1 = cb - p1.astype(jnp.float32)
        p2 = r1.astype(jnp.bfloat16)
        p3 = (r1 - p2.astype(jnp.float32)).astype(jnp.bfloat16)
        packed = jnp.concatenate([p1, p2, p3], axis=1)
        g3 = jax.lax.dot_general(
            onehot, packed, (((1,), (0,)), ((), ())),
            preferred_element_type=jnp.float32,
        )
        g = (g3[:, :_D] + g3[:, _D:2 * _D]) + g3[:, 2 * _D:]
        r = r - g
        q = q + g
        idx_cols.append(idx)
    q_ref[...] = q
    idx_ref[...] = jnp.concatenate(idx_cols, axis=1)


def kernel(z, codebooks):
    grid = (_B // _BLK,)
    q, idx = pl.pallas_call(
        _rvq_body,
        grid=grid,
        in_specs=[
            pl.BlockSpec((_BLK, _D), lambda i: (i, 0)),
            pl.BlockSpec((_NCB, _K, _D), lambda i: (0, 0, 0)),
        ],
        out_specs=[
            pl.BlockSpec((_BLK, _D), lambda i: (i, 0)),
            pl.BlockSpec((_BLK, _NCB), lambda i: (i, 0)),
        ],
        out_shape=[
            jax.ShapeDtypeStruct((_B, _D), jnp.float32),
            jax.ShapeDtypeStruct((_B, _NCB), jnp.int32),
        ],
    )(z, codebooks)
    loss = jnp.zeros((), dtype=jnp.float32)
    return q, loss, idx.astype(jnp.int64)
